# baseline (speedup 1.0000x reference)
import functools

import jax
import jax.numpy as jnp
from jax.experimental import pallas as pl
from jax.experimental.pallas import tpu as pltpu

_BN = 512
_SUB = 256


def _layer_kernel(a0_ref, a1_ref, x_ref, nrm_ref, watt_ref, w_ref, b_ref,
                  o_ref):
    x = x_ref[...]
    watt = watt_ref[...]
    w = w_ref[...]
    bias = b_ref[...]
    bn = o_ref.shape[0]
    for h in range(bn // _SUB):
        lo = h * _SUB
        ab = jnp.concatenate(
            [a0_ref[0, lo:lo + _SUB, :].astype(jnp.bfloat16),
             a1_ref[0, lo:lo + _SUB, :].astype(jnp.bfloat16)], axis=0)
        tb = jnp.dot(ab, x, preferred_element_type=jnp.float32)
        t0 = tb[0:_SUB]
        t1 = tb[_SUB:2 * _SUB]
        nrm = nrm_ref[lo:lo + _SUB]
        t0 = t0 / nrm[:, 0:1]
        t1 = t1 / nrm[:, 1:2]
        u = t0 - t1
        d = jnp.dot(u, watt, preferred_element_type=jnp.float32)
        p = 1.0 / (1.0 + jnp.exp(-d))
        comb = t1 + p * u
        out = jnp.dot(comb, w, preferred_element_type=jnp.float32)
        out = jnp.maximum(out + bias, 0.0)
        o_ref[lo:lo + _SUB, :] = out.astype(o_ref.dtype)


def _layer(x, motifs_all, nrm_t, w_att, w, b, out_dtype, *,
           interpret=False):
    n = x.shape[0]
    d_in = x.shape[1]
    d_out = w.shape[1]
    m = nrm_t.shape[1]
    bn = _BN
    grid = (n // bn,)
    return pl.pallas_call(
        _layer_kernel,
        grid=grid,
        in_specs=[
            pl.BlockSpec((1, bn, n), lambda i: (0, i, 0)),
            pl.BlockSpec((1, bn, n), lambda i: (1, i, 0)),
            pl.BlockSpec((n, d_in), lambda i: (0, 0)),
            pl.BlockSpec((bn, m), lambda i: (i, 0)),
            pl.BlockSpec((d_in, 1), lambda i: (0, 0)),
            pl.BlockSpec((d_in, d_out), lambda i: (0, 0)),
            pl.BlockSpec((1, d_out), lambda i: (0, 0)),
        ],
        out_specs=pl.BlockSpec((bn, d_out), lambda i: (i, 0)),
        out_shape=jax.ShapeDtypeStruct((n, d_out), out_dtype),
        compiler_params=pltpu.CompilerParams(
            dimension_semantics=("arbitrary",)),
        interpret=interpret,
    )(motifs_all, motifs_all, x, nrm_t, w_att, w, b)


@jax.jit
def kernel(x, motifs_all, motifs_num, w_att0, b_att0, W0, b0,
           w_att1, b_att1, W1, b1):
    del b_att0, b_att1
    nrm_t = motifs_num.T
    h = _layer(x.astype(jnp.bfloat16), motifs_all, nrm_t,
               w_att0, W0, b0.reshape(1, -1), jnp.bfloat16)
    return _layer(h, motifs_all, nrm_t,
                  w_att1, W1, b1.reshape(1, -1), jnp.float32)


# speedup vs baseline: 1.0534x; 1.0534x over previous
import functools

import jax
import jax.numpy as jnp
from jax.experimental import pallas as pl
from jax.experimental.pallas import tpu as pltpu

_BN = 256
_SUB = 128


def _layer_kernel(a0_ref, a1_ref, x_ref, nrm_ref, watt_ref, w_ref, b_ref,
                  o_ref):
    x = x_ref[...]
    watt = watt_ref[...]
    w = w_ref[...]
    bias = b_ref[...]
    bn = o_ref.shape[0]
    for h in range(bn // _SUB):
        lo = h * _SUB
        t0 = jnp.dot(a0_ref[0, lo:lo + _SUB, :].astype(jnp.bfloat16), x,
                     preferred_element_type=jnp.float32)
        t1 = jnp.dot(a1_ref[0, lo:lo + _SUB, :].astype(jnp.bfloat16), x,
                     preferred_element_type=jnp.float32)
        nrm = nrm_ref[lo:lo + _SUB]
        t0 = t0 / nrm[:, 0:1]
        t1 = t1 / nrm[:, 1:2]
        u = t0 - t1
        d = jnp.dot(u, watt, preferred_element_type=jnp.float32)
        p = 1.0 / (1.0 + jnp.exp(-d))
        comb = t1 + p * u
        out = jnp.dot(comb, w, preferred_element_type=jnp.float32)
        out = jnp.maximum(out + bias, 0.0)
        o_ref[lo:lo + _SUB, :] = out.astype(o_ref.dtype)


def _layer(x, motifs_all, nrm_t, w_att, w, b, out_dtype, *,
           interpret=False):
    n = x.shape[0]
    d_in = x.shape[1]
    d_out = w.shape[1]
    m = nrm_t.shape[1]
    bn = _BN
    grid = (n // bn,)
    return pl.pallas_call(
        _layer_kernel,
        grid=grid,
        in_specs=[
            pl.BlockSpec((1, bn, n), lambda i: (0, i, 0)),
            pl.BlockSpec((1, bn, n), lambda i: (1, i, 0)),
            pl.BlockSpec((n, d_in), lambda i: (0, 0)),
            pl.BlockSpec((bn, m), lambda i: (i, 0)),
            pl.BlockSpec((d_in, 1), lambda i: (0, 0)),
            pl.BlockSpec((d_in, d_out), lambda i: (0, 0)),
            pl.BlockSpec((1, d_out), lambda i: (0, 0)),
        ],
        out_specs=pl.BlockSpec((bn, d_out), lambda i: (i, 0)),
        out_shape=jax.ShapeDtypeStruct((n, d_out), out_dtype),
        compiler_params=pltpu.CompilerParams(
            dimension_semantics=("arbitrary",)),
        interpret=interpret,
    )(motifs_all, motifs_all, x, nrm_t, w_att, w, b)


@jax.jit
def kernel(x, motifs_all, motifs_num, w_att0, b_att0, W0, b0,
           w_att1, b_att1, W1, b1):
    del b_att0, b_att1
    nrm_t = motifs_num.T
    h = _layer(x.astype(jnp.bfloat16), motifs_all, nrm_t,
               w_att0, W0, b0.reshape(1, -1), jnp.bfloat16)
    return _layer(h, motifs_all, nrm_t,
                  w_att1, W1, b1.reshape(1, -1), jnp.float32)


# speedup vs baseline: 1.0845x; 1.0295x over previous
import functools

import jax
import jax.numpy as jnp
from jax.experimental import pallas as pl
from jax.experimental.pallas import tpu as pltpu

_BN = 512
_SUB = 128


def _layer_kernel(a0_ref, a1_ref, x_ref, nrm_ref, watt_ref, w_ref, b_ref,
                  o_ref):
    x = x_ref[...]
    watt = watt_ref[...]
    w = w_ref[...]
    bias = b_ref[...]
    bn = o_ref.shape[0]
    for h in range(bn // _SUB):
        lo = h * _SUB
        t0 = jnp.dot(a0_ref[0, lo:lo + _SUB, :].astype(jnp.bfloat16), x,
                     preferred_element_type=jnp.float32)
        t1 = jnp.dot(a1_ref[0, lo:lo + _SUB, :].astype(jnp.bfloat16), x,
                     preferred_element_type=jnp.float32)
        nrm = nrm_ref[lo:lo + _SUB]
        t0 = t0 / nrm[:, 0:1]
        t1 = t1 / nrm[:, 1:2]
        u = t0 - t1
        d = jnp.dot(u, watt, preferred_element_type=jnp.float32)
        p = 1.0 / (1.0 + jnp.exp(-d))
        comb = t1 + p * u
        out = jnp.dot(comb, w, preferred_element_type=jnp.float32)
        out = jnp.maximum(out + bias, 0.0)
        o_ref[lo:lo + _SUB, :] = out.astype(o_ref.dtype)


def _layer(x, motifs_all, nrm_t, w_att, w, b, out_dtype, *,
           interpret=False):
    n = x.shape[0]
    d_in = x.shape[1]
    d_out = w.shape[1]
    m = nrm_t.shape[1]
    bn = _BN
    grid = (n // bn,)
    return pl.pallas_call(
        _layer_kernel,
        grid=grid,
        in_specs=[
            pl.BlockSpec((1, bn, n), lambda i: (0, i, 0)),
            pl.BlockSpec((1, bn, n), lambda i: (1, i, 0)),
            pl.BlockSpec((n, d_in), lambda i: (0, 0)),
            pl.BlockSpec((bn, m), lambda i: (i, 0)),
            pl.BlockSpec((d_in, 1), lambda i: (0, 0)),
            pl.BlockSpec((d_in, d_out), lambda i: (0, 0)),
            pl.BlockSpec((1, d_out), lambda i: (0, 0)),
        ],
        out_specs=pl.BlockSpec((bn, d_out), lambda i: (i, 0)),
        out_shape=jax.ShapeDtypeStruct((n, d_out), out_dtype),
        compiler_params=pltpu.CompilerParams(
            dimension_semantics=("arbitrary",)),
        interpret=interpret,
    )(motifs_all, motifs_all, x, nrm_t, w_att, w, b)


@jax.jit
def kernel(x, motifs_all, motifs_num, w_att0, b_att0, W0, b0,
           w_att1, b_att1, W1, b1):
    del b_att0, b_att1
    nrm_t = motifs_num.T
    h = _layer(x.astype(jnp.bfloat16), motifs_all, nrm_t,
               w_att0, W0, b0.reshape(1, -1), jnp.bfloat16)
    return _layer(h, motifs_all, nrm_t,
                  w_att1, W1, b1.reshape(1, -1), jnp.float32)


# speedup vs baseline: 1.0943x; 1.0091x over previous
import functools

import jax
import jax.numpy as jnp
from jax.experimental import pallas as pl
from jax.experimental.pallas import tpu as pltpu

_BN = 512
_SUB = 256


def _layer_kernel(a0_ref, a1_ref, x_ref, nrm_ref, watt_ref, w_ref, b_ref,
                  o_ref):
    x = x_ref[...]
    watt = watt_ref[...]
    w = w_ref[...]
    bias = b_ref[...]
    bn = o_ref.shape[0]
    for h in range(bn // _SUB):
        lo = h * _SUB
        t0 = jnp.dot(a0_ref[0, lo:lo + _SUB, :].astype(jnp.bfloat16), x,
                     preferred_element_type=jnp.float32)
        t1 = jnp.dot(a1_ref[0, lo:lo + _SUB, :].astype(jnp.bfloat16), x,
                     preferred_element_type=jnp.float32)
        nrm = nrm_ref[lo:lo + _SUB]
        t0 = t0 / nrm[:, 0:1]
        t1 = t1 / nrm[:, 1:2]
        u = t0 - t1
        d = jnp.dot(u, watt, preferred_element_type=jnp.float32)
        p = 1.0 / (1.0 + jnp.exp(-d))
        comb = t1 + p * u
        out = jnp.dot(comb, w, preferred_element_type=jnp.float32)
        out = jnp.maximum(out + bias, 0.0)
        o_ref[lo:lo + _SUB, :] = out.astype(o_ref.dtype)


def _layer(x, motifs_all, nrm_t, w_att, w, b, out_dtype, *,
           interpret=False):
    n = x.shape[0]
    d_in = x.shape[1]
    d_out = w.shape[1]
    m = nrm_t.shape[1]
    bn = _BN
    grid = (n // bn,)
    return pl.pallas_call(
        _layer_kernel,
        grid=grid,
        in_specs=[
            pl.BlockSpec((1, bn, n), lambda i: (0, i, 0)),
            pl.BlockSpec((1, bn, n), lambda i: (1, i, 0)),
            pl.BlockSpec((n, d_in), lambda i: (0, 0)),
            pl.BlockSpec((bn, m), lambda i: (i, 0)),
            pl.BlockSpec((d_in, 1), lambda i: (0, 0)),
            pl.BlockSpec((d_in, d_out), lambda i: (0, 0)),
            pl.BlockSpec((1, d_out), lambda i: (0, 0)),
        ],
        out_specs=pl.BlockSpec((bn, d_out), lambda i: (i, 0)),
        out_shape=jax.ShapeDtypeStruct((n, d_out), out_dtype),
        compiler_params=pltpu.CompilerParams(
            dimension_semantics=("arbitrary",)),
        interpret=interpret,
    )(motifs_all, motifs_all, x, nrm_t, w_att, w, b)


@jax.jit
def kernel(x, motifs_all, motifs_num, w_att0, b_att0, W0, b0,
           w_att1, b_att1, W1, b1):
    del b_att0, b_att1
    nrm_t = motifs_num.T
    h = _layer(x.astype(jnp.bfloat16), motifs_all, nrm_t,
               w_att0, W0, b0.reshape(1, -1), jnp.bfloat16)
    return _layer(h, motifs_all, nrm_t,
                  w_att1, W1, b1.reshape(1, -1), jnp.float32)
